# baseline (speedup 1.0000x reference)
import jax
import jax.numpy as jnp
from jax import lax
from jax.experimental import pallas as pl
from jax.experimental.pallas import tpu as pltpu
from jax.experimental.pallas import tpu_sc as plsc

_info = plsc.get_sparse_core_info()
_NC, _NS = _info.num_cores, _info.num_subcores
_NW = _NC * _NS

_BAG = 26
_BATCH = 4096
_ROWS = _BATCH * _BAG
_D = 128
_CHUNK = 208
_PER_W = _ROWS // _NW
_NCHUNK = _PER_W // _CHUNK
_NBUF = 4
_LAG = 2


def _sc_gather(table, idx1):
  mesh = plsc.VectorSubcoreMesh(core_axis_name="c", subcore_axis_name="s")

  def body(table_hbm, idx_hbm, out_hbm, idx_v, rows_v, gsems, ssems):
    wid = lax.axis_index("s") * _NC + lax.axis_index("c")
    base = wid * _PER_W
    pltpu.sync_copy(idx_hbm.at[pl.ds(base, _PER_W)], idx_v)

    def gather(c, b):
      return pltpu.make_async_copy(
          table_hbm.at[idx_v.at[pl.ds(c * _CHUNK, _CHUNK)]],
          rows_v.at[b], gsems.at[b])

    def store(c, b):
      return pltpu.make_async_copy(
          rows_v.at[b], out_hbm.at[pl.ds(base + c * _CHUNK, _CHUNK)],
          ssems.at[b])

    for c in range(_NCHUNK + _LAG):
      b = c % _NBUF
      if c < _NCHUNK:
        if c >= _NBUF:
          store(c - _NBUF, b).wait()
        gather(c, b).start()
      d = c - _LAG
      if d >= 0:
        bd = d % _NBUF
        gather(d, bd).wait()
        store(d, bd).start()
    for d in range(_NCHUNK - _NBUF, _NCHUNK):
      store(d, d % _NBUF).wait()

  f = pl.kernel(
      body,
      out_type=jax.ShapeDtypeStruct((_ROWS, _D), jnp.float32),
      mesh=mesh,
      scratch_types=[
          pltpu.VMEM((_PER_W,), jnp.int32),
          pltpu.VMEM((_NBUF, _CHUNK, _D), jnp.float32),
          pltpu.SemaphoreType.DMA((_NBUF,)),
          pltpu.SemaphoreType.DMA((_NBUF,)),
      ],
  )
  return f(table, idx1)


def kernel(inputs, embedding):
  idx1 = inputs.astype(jnp.int32).T.reshape(_ROWS)
  out = _sc_gather(embedding, idx1)
  return out.reshape(_BAG, _BATCH, _D).transpose(1, 0, 2)


# speedup vs baseline: 1.0051x; 1.0051x over previous
import jax
import jax.numpy as jnp
from jax import lax
from jax.experimental import pallas as pl
from jax.experimental.pallas import tpu as pltpu
from jax.experimental.pallas import tpu_sc as plsc

_info = plsc.get_sparse_core_info()
_NC, _NS = _info.num_cores, _info.num_subcores
_NW = _NC * _NS

_BAG = 26
_BATCH = 4096
_ROWS = _BATCH * _BAG
_D = 128
_CHUNK = 256
_PER_W = _ROWS // _NW
_NCHUNK = _PER_W // _CHUNK
_NBUF = 3
_LAG = 2


def _sc_gather(table, idx1):
  mesh = plsc.VectorSubcoreMesh(core_axis_name="c", subcore_axis_name="s")

  def body(table_hbm, idx_hbm, out_hbm, idx_v, rows_v, gsems, ssems):
    wid = lax.axis_index("s") * _NC + lax.axis_index("c")
    base = wid * _PER_W
    pltpu.sync_copy(idx_hbm.at[pl.ds(base, _PER_W)], idx_v)

    def gather(c, b):
      return pltpu.make_async_copy(
          table_hbm.at[idx_v.at[pl.ds(c * _CHUNK, _CHUNK)]],
          rows_v.at[b], gsems.at[b])

    def store(c, b):
      return pltpu.make_async_copy(
          rows_v.at[b], out_hbm.at[pl.ds(base + c * _CHUNK, _CHUNK)],
          ssems.at[b])

    for c in range(_NCHUNK + _LAG):
      b = c % _NBUF
      if c < _NCHUNK:
        if c >= _NBUF:
          store(c - _NBUF, b).wait()
        gather(c, b).start()
      d = c - _LAG
      if d >= 0:
        bd = d % _NBUF
        gather(d, bd).wait()
        store(d, bd).start()
    for d in range(_NCHUNK - _NBUF, _NCHUNK):
      store(d, d % _NBUF).wait()

  f = pl.kernel(
      body,
      out_type=jax.ShapeDtypeStruct((_ROWS, _D), jnp.float32),
      mesh=mesh,
      scratch_types=[
          pltpu.VMEM((_PER_W,), jnp.int32),
          pltpu.VMEM((_NBUF, _CHUNK, _D), jnp.float32),
          pltpu.SemaphoreType.DMA((_NBUF,)),
          pltpu.SemaphoreType.DMA((_NBUF,)),
      ],
  )
  return f(table, idx1)


def kernel(inputs, embedding):
  idx1 = inputs.astype(jnp.int32).T.reshape(_ROWS)
  out = _sc_gather(embedding, idx1)
  return out.reshape(_BAG, _BATCH, _D).transpose(1, 0, 2)
